# baseline (speedup 1.0000x reference)
import jax
import jax.numpy as jnp
from jax import lax
from jax.experimental import pallas as pl
from jax.experimental.pallas import tpu as pltpu
from jax.experimental.pallas import tpu_sc as plsc

VOCAB = 1000000
EMBED = 64
B = 16384
L = 50
LANES = 16

NC = 2
NS = 16
NW = NC * NS

C = 8
BPW = B // NW
NCHUNK = BPW // C
ROWS = C * L
GPB = 4
UPAD = 16

_SPLITS = []
_off = 0
while _off < ROWS:
    _n = min(128, ROWS - _off)
    _SPLITS.append((_off, _n))
    _off += _n


def _sc_kernel(ctr_hbm, ctx_hbm, ev_hbm, eu_hbm, out_hbm,
               ctr_idx, ctx_idx,
               u0, u1, v0, v1, o0, o1,
               sem0, sem1):
    wid = lax.axis_index("s") * NC + lax.axis_index("c")
    base = wid * BPW

    iota = lax.iota(jnp.int32, LANES)

    def descriptors(h, ub, vb, sem):
        ioff = pl.multiple_of(h * ROWS, 8)
        out = []
        for (off, n) in _SPLITS:
            out.append(pltpu.make_async_copy(
                eu_hbm.at[ctx_idx.at[pl.ds(ioff + off, n)]],
                ub.at[pl.ds(off, n)], sem))
        out.append(pltpu.make_async_copy(
            ev_hbm.at[ctr_idx.at[pl.ds(pl.multiple_of(h * C, 8), C)]],
            vb, sem))
        return out

    def fire(h, ub, vb, sem):
        for d in descriptors(h, ub, vb, sem):
            d.start()

    def drain(h, ub, vb, sem):
        for d in descriptors(h, ub, vb, sem):
            d.wait()

    def compute(g, ub, vb, ob):
        def b_body(b, _):
            vvecs = [vb[b, pl.ds(k * LANES, LANES)] for k in range(4)]
            row0 = b * L
            rows = [row0 + grp * LANES + iota for grp in range(GPB)]
            accs = [jnp.zeros((LANES,), jnp.float32) for _ in range(GPB)]
            for e in range(EMBED):
                vsplat = jnp.take(vvecs[e // LANES],
                                  jnp.full((LANES,), e % LANES, jnp.int32))
                e_vec = jnp.full((LANES,), e, jnp.int32)
                for grp in range(GPB):
                    uvals = plsc.load_gather(ub, [rows[grp], e_vec])
                    accs[grp] = accs[grp] + uvals * vsplat
            for grp in range(GPB):
                lane_l = grp * LANES + iota
                plsc.store_scatter(ob, [lane_l, jnp.full((LANES,), b, jnp.int32)],
                                   accs[grp], mask=lane_l < L)
            return ()

        lax.fori_loop(0, C, b_body, (), unroll=False)
        pltpu.sync_copy(
            ob, out_hbm.at[:, pl.ds(pl.multiple_of(base + g * C, 8), C)])

    pltpu.sync_copy(ctr_hbm.at[pl.ds(base, BPW)], ctr_idx)
    pltpu.sync_copy(ctx_hbm.at[pl.ds(base * L, BPW * L)], ctx_idx)

    fire(0, u0, v0, sem0)

    def pair_body(k, _):
        g0 = 2 * k
        fire(g0 + 1, u1, v1, sem1)
        drain(g0, u0, v0, sem0)
        compute(g0, u0, v0, o0)

        g1 = 2 * k + 1

        @pl.when(g1 + 1 < NCHUNK)
        def _():
            fire(g1 + 1, u0, v0, sem0)

        drain(g1, u1, v1, sem1)
        compute(g1, u1, v1, o1)
        return ()

    lax.fori_loop(0, NCHUNK // 2, pair_body, (), unroll=False)


@jax.jit
def _run(centers, contexts_negatives, embed_v, embed_u):
    kfn = pl.kernel(
        _sc_kernel,
        out_type=jax.ShapeDtypeStruct((L, B), jnp.float32),
        mesh=plsc.VectorSubcoreMesh(core_axis_name="c", subcore_axis_name="s"),
        scratch_types=[
            pltpu.VMEM((BPW,), jnp.int32),
            pltpu.VMEM((BPW * L,), jnp.int32),
            pltpu.VMEM((ROWS + UPAD, EMBED), jnp.float32),
            pltpu.VMEM((ROWS + UPAD, EMBED), jnp.float32),
            pltpu.VMEM((C, EMBED), jnp.float32),
            pltpu.VMEM((C, EMBED), jnp.float32),
            pltpu.VMEM((L, C), jnp.float32),
            pltpu.VMEM((L, C), jnp.float32),
            pltpu.SemaphoreType.DMA,
            pltpu.SemaphoreType.DMA,
        ],
        compiler_params=pltpu.CompilerParams(use_tc_tiling_on_sc=False,
                                             needs_layout_passes=False),
    )
    out_lb = kfn(centers.reshape(B), contexts_negatives.reshape(B * L),
                 embed_v, embed_u)
    return jnp.transpose(out_lb)[:, None, :]


def kernel(centers, contexts_negatives, embed_v, embed_u):
    return _run(centers, contexts_negatives, embed_v, embed_u)


# speedup vs baseline: 1.7034x; 1.7034x over previous
import jax
import jax.numpy as jnp
from jax import lax
from jax.experimental import pallas as pl
from jax.experimental.pallas import tpu as pltpu
from jax.experimental.pallas import tpu_sc as plsc

VOCAB = 1000000
EMBED = 64
B = 16384
L = 50
LANES = 16

NC = 2
NS = 16
NW = NC * NS

C = 8
BPW = B // NW
NCHUNK = BPW // C
ROWS = C * L
GPB = 4
UPAD = 16

_SPLITS = []
_off = 0
while _off < ROWS:
    _n = min(128, ROWS - _off)
    _SPLITS.append((_off, _n))
    _off += _n


def _sc_kernel(ctr_hbm, ctx_hbm, ev_hbm, eu_hbm, out_hbm,
               ctr_idx, ctx_idx,
               u0, u1, v0, v1, o0, o1,
               sem0, sem1):
    wid = lax.axis_index("s") * NC + lax.axis_index("c")
    base = wid * BPW

    iota = lax.iota(jnp.int32, LANES)

    def descriptors(h, ub, vb, sem):
        ioff = pl.multiple_of(h * ROWS, 8)
        out = []
        for (off, n) in _SPLITS:
            out.append(pltpu.make_async_copy(
                eu_hbm.at[ctx_idx.at[pl.ds(ioff + off, n)]],
                ub.at[pl.ds(off, n)], sem))
        out.append(pltpu.make_async_copy(
            ev_hbm.at[ctr_idx.at[pl.ds(pl.multiple_of(h * C, 8), C)]],
            vb, sem))
        return out

    def fire(h, ub, vb, sem):
        for d in descriptors(h, ub, vb, sem):
            d.start()

    def drain(h, ub, vb, sem):
        for d in descriptors(h, ub, vb, sem):
            d.wait()

    rots = [jnp.bitwise_and(iota + e, LANES - 1) for e in range(LANES)]

    def compute(g, ub, vb, ob):
        def b_body(b, _):
            vvecs = [vb[b, pl.ds(k * LANES, LANES)] for k in range(4)]
            row0 = b * L
            rows = [row0 + grp * LANES + iota for grp in range(GPB)]
            accs = [jnp.zeros((LANES,), jnp.float32) for _ in range(GPB)]
            for eblk in range(EMBED // LANES):
                for esub in range(LANES):
                    vrot = jnp.take(vvecs[eblk], rots[esub])
                    col = eblk * LANES + rots[esub]
                    for grp in range(GPB):
                        uvals = plsc.load_gather(ub, [rows[grp], col])
                        accs[grp] = accs[grp] + uvals * vrot
            for grp in range(GPB):
                lane_l = grp * LANES + iota
                plsc.store_scatter(ob, [lane_l, jnp.full((LANES,), b, jnp.int32)],
                                   accs[grp], mask=lane_l < L)
            return ()

        lax.fori_loop(0, C, b_body, (), unroll=False)
        pltpu.sync_copy(
            ob, out_hbm.at[:, pl.ds(pl.multiple_of(base + g * C, 8), C)])

    pltpu.sync_copy(ctr_hbm.at[pl.ds(base, BPW)], ctr_idx)
    pltpu.sync_copy(ctx_hbm.at[pl.ds(base * L, BPW * L)], ctx_idx)

    fire(0, u0, v0, sem0)

    def pair_body(k, _):
        g0 = 2 * k
        fire(g0 + 1, u1, v1, sem1)
        drain(g0, u0, v0, sem0)
        compute(g0, u0, v0, o0)

        g1 = 2 * k + 1

        @pl.when(g1 + 1 < NCHUNK)
        def _():
            fire(g1 + 1, u0, v0, sem0)

        drain(g1, u1, v1, sem1)
        compute(g1, u1, v1, o1)
        return ()

    lax.fori_loop(0, NCHUNK // 2, pair_body, (), unroll=False)


@jax.jit
def _run(centers, contexts_negatives, embed_v, embed_u):
    kfn = pl.kernel(
        _sc_kernel,
        out_type=jax.ShapeDtypeStruct((L, B), jnp.float32),
        mesh=plsc.VectorSubcoreMesh(core_axis_name="c", subcore_axis_name="s"),
        scratch_types=[
            pltpu.VMEM((BPW,), jnp.int32),
            pltpu.VMEM((BPW * L,), jnp.int32),
            pltpu.VMEM((ROWS + UPAD, EMBED), jnp.float32),
            pltpu.VMEM((ROWS + UPAD, EMBED), jnp.float32),
            pltpu.VMEM((C, EMBED), jnp.float32),
            pltpu.VMEM((C, EMBED), jnp.float32),
            pltpu.VMEM((L, C), jnp.float32),
            pltpu.VMEM((L, C), jnp.float32),
            pltpu.SemaphoreType.DMA,
            pltpu.SemaphoreType.DMA,
        ],
        compiler_params=pltpu.CompilerParams(use_tc_tiling_on_sc=False,
                                             needs_layout_passes=False),
    )
    out_lb = kfn(centers.reshape(B), contexts_negatives.reshape(B * L),
                 embed_v, embed_u)
    return jnp.transpose(out_lb)[:, None, :]


def kernel(centers, contexts_negatives, embed_v, embed_u):
    return _run(centers, contexts_negatives, embed_v, embed_u)


# speedup vs baseline: 1.7120x; 1.0050x over previous
import jax
import jax.numpy as jnp
from jax import lax
from jax.experimental import pallas as pl
from jax.experimental.pallas import tpu as pltpu
from jax.experimental.pallas import tpu_sc as plsc

VOCAB = 1000000
EMBED = 64
B = 16384
L = 50
LANES = 16

NC = 2
NS = 16
NW = NC * NS

C = 8
BPW = B // NW
NCHUNK = BPW // C
ROWS = C * L
GPB = 4
UPAD = 16

_SPLITS = []
_off = 0
while _off < ROWS:
    _n = min(128, ROWS - _off)
    _SPLITS.append((_off, _n))
    _off += _n


def _sc_kernel(ctr_hbm, ctx_hbm, ev_hbm, eu_hbm, out_hbm,
               ctr2d, ctr_idx, ctxT, ctx_idx,
               u0, u1, v0, v1, o0, o1,
               sem0, sem1):
    wid = lax.axis_index("s") * NC + lax.axis_index("c")
    base = wid * BPW

    iota = lax.iota(jnp.int32, LANES)

    def descriptors(h, ub, vb, sem):
        ioff = pl.multiple_of(h * ROWS, 8)
        out = []
        for (off, n) in _SPLITS:
            out.append(pltpu.make_async_copy(
                eu_hbm.at[ctx_idx.at[pl.ds(ioff + off, n)]],
                ub.at[pl.ds(off, n)], sem))
        out.append(pltpu.make_async_copy(
            ev_hbm.at[ctr_idx.at[pl.ds(pl.multiple_of(h * C, 8), C)]],
            vb, sem))
        return out

    def fire(h, ub, vb, sem):
        for d in descriptors(h, ub, vb, sem):
            d.start()

    def drain(h, ub, vb, sem):
        for d in descriptors(h, ub, vb, sem):
            d.wait()

    rots = [jnp.bitwise_and(iota + e, LANES - 1) for e in range(LANES)]

    def compute(g, ub, vb, ob):
        def b_body(b, _):
            vvecs = [vb[b, pl.ds(k * LANES, LANES)] for k in range(4)]
            row0 = b * L
            rows = [row0 + grp * LANES + iota for grp in range(GPB)]
            accs = [jnp.zeros((LANES,), jnp.float32) for _ in range(GPB)]
            for eblk in range(EMBED // LANES):
                for esub in range(LANES):
                    vrot = jnp.take(vvecs[eblk], rots[esub])
                    col = eblk * LANES + rots[esub]
                    for grp in range(GPB):
                        uvals = plsc.load_gather(ub, [rows[grp], col])
                        accs[grp] = accs[grp] + uvals * vrot
            for grp in range(GPB):
                lane_l = grp * LANES + iota
                plsc.store_scatter(ob, [lane_l, jnp.full((LANES,), b, jnp.int32)],
                                   accs[grp], mask=lane_l < L)
            return ()

        lax.fori_loop(0, C, b_body, (), unroll=False)
        pltpu.sync_copy(
            ob, out_hbm.at[:, pl.ds(pl.multiple_of(base + g * C, 8), C)])

    pltpu.sync_copy(ctr_hbm.at[:, pl.ds(base, BPW)], ctr2d)
    pltpu.sync_copy(ctx_hbm.at[:, pl.ds(base, BPW)], ctxT)

    for k in range(BPW // LANES):
        ctr_idx[pl.ds(k * LANES, LANES)] = ctr2d[0, pl.ds(k * LANES, LANES)]

    iota_l = iota * L

    def t_body(k, _):
        kbase = k * LANES * L
        for l in range(L):
            vec = ctxT[l, pl.ds(k * LANES, LANES)]
            plsc.store_scatter(ctx_idx, [kbase + iota_l + l], vec)
        return ()

    lax.fori_loop(0, BPW // LANES, t_body, (), unroll=False)

    fire(0, u0, v0, sem0)

    def pair_body(k, _):
        g0 = 2 * k
        fire(g0 + 1, u1, v1, sem1)
        drain(g0, u0, v0, sem0)
        compute(g0, u0, v0, o0)

        g1 = 2 * k + 1

        @pl.when(g1 + 1 < NCHUNK)
        def _():
            fire(g1 + 1, u0, v0, sem0)

        drain(g1, u1, v1, sem1)
        compute(g1, u1, v1, o1)
        return ()

    lax.fori_loop(0, NCHUNK // 2, pair_body, (), unroll=False)


@jax.jit
def _run(centers, contexts_negatives, embed_v, embed_u):
    kfn = pl.kernel(
        _sc_kernel,
        out_type=jax.ShapeDtypeStruct((L, B), jnp.float32),
        mesh=plsc.VectorSubcoreMesh(core_axis_name="c", subcore_axis_name="s"),
        scratch_types=[
            pltpu.VMEM((1, BPW), jnp.int32),
            pltpu.VMEM((BPW,), jnp.int32),
            pltpu.VMEM((L, BPW), jnp.int32),
            pltpu.VMEM((BPW * L,), jnp.int32),
            pltpu.VMEM((ROWS + UPAD, EMBED), jnp.float32),
            pltpu.VMEM((ROWS + UPAD, EMBED), jnp.float32),
            pltpu.VMEM((C, EMBED), jnp.float32),
            pltpu.VMEM((C, EMBED), jnp.float32),
            pltpu.VMEM((L, C), jnp.float32),
            pltpu.VMEM((L, C), jnp.float32),
            pltpu.SemaphoreType.DMA,
            pltpu.SemaphoreType.DMA,
        ],
        compiler_params=pltpu.CompilerParams(use_tc_tiling_on_sc=False,
                                             needs_layout_passes=False),
    )
    out_lb = kfn(jnp.transpose(centers), jnp.transpose(contexts_negatives),
                 embed_v, embed_u)
    return jnp.transpose(out_lb)[:, None, :]


def kernel(centers, contexts_negatives, embed_v, embed_u):
    return _run(centers, contexts_negatives, embed_v, embed_u)
